# baseline (speedup 1.0000x reference)
import jax
import jax.numpy as jnp
from jax.experimental import pallas as pl
from jax.experimental.pallas import tpu as pltpu

_MAX_DIS = 15
_B = 4
_N = 8192
_L = 2048
_LB = 128
_PS = 256
_FLUSH = 16


def _cdist_count_kernel(ah_ref, al_ref, lh_ref, ll_ref, out_ref):
    lh = lh_ref[...]
    ll = ll_ref[...]
    one = jnp.bfloat16(1.0)
    zero = jnp.bfloat16(0.0)
    dims = (((0,), (0,)), ((), ()))
    n_seg = _N // _PS // _FLUSH

    for b in range(_B):
        accs = [jnp.zeros((1, _LB), jnp.float32) for _ in range(_MAX_DIS)]
        for seg in range(n_seg):

            def chunk_body(ci, acc16):
                c = seg * _FLUSH + ci
                off = pl.multiple_of(c * _PS, _PS)
                pah = ah_ref[b, :, pl.ds(off, _PS)]
                pal = al_ref[b, :, pl.ds(off, _PS)]
                d2 = (
                    jax.lax.dot_general(pah, lh, dims,
                                        preferred_element_type=jnp.float32)
                    + jax.lax.dot_general(pah, ll, dims,
                                          preferred_element_type=jnp.float32)
                    + jax.lax.dot_general(pal, lh, dims,
                                          preferred_element_type=jnp.float32)
                )
                e16 = jnp.ceil(d2).astype(jnp.bfloat16)
                new = list(acc16)
                for j in range(_PS // 32):
                    ej = e16[j * 32:(j + 1) * 32]
                    for t in range(_MAX_DIS):
                        thr2 = jnp.bfloat16((t + 1) * (t + 1))
                        m = jnp.where(ej <= thr2, one, zero)
                        m = m[0:16] + m[16:32]
                        new[t] = new[t] + m
                return tuple(new)

            acc16_init = tuple(
                jnp.zeros((16, _LB), jnp.bfloat16) for _ in range(_MAX_DIS)
            )
            acc16 = jax.lax.fori_loop(0, _FLUSH, chunk_body, acc16_init)
            for t in range(_MAX_DIS):
                s = jnp.sum(acc16[t].astype(jnp.float32), axis=0,
                            keepdims=True)
                accs[t] = accs[t] + s
        for t in range(_MAX_DIS):
            out_ref[b, t:t + 1, :] = accs[t]


def _trunc_bf16_exact(x):
    u = jax.lax.bitcast_convert_type(x, jnp.uint32)
    return jax.lax.bitcast_convert_type(u & jnp.uint32(0xFFFF0000),
                                        jnp.float32)


def _split_bf16_exact(x):
    hi = _trunc_bf16_exact(x)
    lo = _trunc_bf16_exact(x - hi)
    return hi, lo


def kernel(pcd, locs):
    pn2 = jnp.sum(pcd * pcd, axis=-1, keepdims=True)
    aug = jnp.concatenate(
        [pcd, pn2, jnp.ones_like(pn2)], axis=-1
    ).transpose(0, 2, 1)
    cl2 = jnp.sum(locs * locs, axis=-1)[None, :]
    laug = jnp.concatenate(
        [-2.0 * locs.T, jnp.ones_like(cl2), cl2], axis=0
    )
    ah, al = _split_bf16_exact(aug)
    lh, ll = _split_bf16_exact(laug)
    res = pl.pallas_call(
        _cdist_count_kernel,
        out_shape=jax.ShapeDtypeStruct((_B, _MAX_DIS, _L), jnp.float32),
        grid=(_L // _LB,),
        in_specs=[
            pl.BlockSpec((_B, 5, _N), lambda i: (0, 0, 0)),
            pl.BlockSpec((_B, 5, _N), lambda i: (0, 0, 0)),
            pl.BlockSpec((5, _LB), lambda i: (0, i)),
            pl.BlockSpec((5, _LB), lambda i: (0, i)),
        ],
        out_specs=pl.BlockSpec((_B, _MAX_DIS, _LB), lambda i: (0, 0, i)),
        compiler_params=pltpu.CompilerParams(
            dimension_semantics=("arbitrary",),
        ),
        name="cdist_count",
    )(ah, al, lh, ll)
    return res.transpose(0, 2, 1)


# speedup vs baseline: 2.2436x; 2.2436x over previous
import jax
import jax.numpy as jnp
from jax.experimental import pallas as pl
from jax.experimental.pallas import tpu as pltpu

_MAX_DIS = 15
_B = 4
_N = 8192
_L = 2048
_LB = 128
_PS = 256
_FLUSH = 16


def _cdist_count_kernel(ah_ref, al_ref, lh_ref, ll_ref, out_ref):
    lh = lh_ref[...]
    ll = ll_ref[...]
    one = jnp.bfloat16(1.0)
    zero = jnp.bfloat16(0.0)
    dims = (((0,), (0,)), ((), ()))
    n_seg = _N // _PS // _FLUSH

    n_chunks = _N // _PS
    for b in range(_B):
        accs = [jnp.zeros((1, _LB), jnp.float32) for _ in range(_MAX_DIS)]
        acc16 = [jnp.zeros((16, _LB), jnp.bfloat16) for _ in range(_MAX_DIS)]
        for c in range(n_chunks):
            pah = ah_ref[b, :, c * _PS:(c + 1) * _PS]
            pal = al_ref[b, :, c * _PS:(c + 1) * _PS]
            d2 = (
                jax.lax.dot_general(pah, lh, dims,
                                    preferred_element_type=jnp.float32)
                + jax.lax.dot_general(pah, ll, dims,
                                      preferred_element_type=jnp.float32)
                + jax.lax.dot_general(pal, lh, dims,
                                      preferred_element_type=jnp.float32)
            )
            e16 = jnp.ceil(d2).astype(jnp.bfloat16)
            for t in range(_MAX_DIS):
                thr2 = jnp.bfloat16((t + 1) * (t + 1))
                m = jnp.where(e16 <= thr2, one, zero)
                m = m[0:128] + m[128:256]
                m = m[0:64] + m[64:128]
                m = m[0:32] + m[32:64]
                m = m[0:16] + m[16:32]
                acc16[t] = acc16[t] + m
            if c % _FLUSH == _FLUSH - 1:
                for t in range(_MAX_DIS):
                    s = jnp.sum(acc16[t].astype(jnp.float32), axis=0,
                                keepdims=True)
                    accs[t] = accs[t] + s
                    acc16[t] = jnp.zeros((16, _LB), jnp.bfloat16)
        for t in range(_MAX_DIS):
            out_ref[b, t:t + 1, :] = accs[t]


def _trunc_bf16_exact(x):
    u = jax.lax.bitcast_convert_type(x, jnp.uint32)
    return jax.lax.bitcast_convert_type(u & jnp.uint32(0xFFFF0000),
                                        jnp.float32)


def _split_bf16_exact(x):
    hi = _trunc_bf16_exact(x)
    lo = _trunc_bf16_exact(x - hi)
    return hi, lo


def kernel(pcd, locs):
    pn2 = jnp.sum(pcd * pcd, axis=-1, keepdims=True)
    aug = jnp.concatenate(
        [pcd, pn2, jnp.ones_like(pn2)], axis=-1
    ).transpose(0, 2, 1)
    cl2 = jnp.sum(locs * locs, axis=-1)[None, :]
    laug = jnp.concatenate(
        [-2.0 * locs.T, jnp.ones_like(cl2), cl2], axis=0
    )
    ah, al = _split_bf16_exact(aug)
    lh, ll = _split_bf16_exact(laug)
    res = pl.pallas_call(
        _cdist_count_kernel,
        out_shape=jax.ShapeDtypeStruct((_B, _MAX_DIS, _L), jnp.float32),
        grid=(_L // _LB,),
        in_specs=[
            pl.BlockSpec((_B, 5, _N), lambda i: (0, 0, 0)),
            pl.BlockSpec((_B, 5, _N), lambda i: (0, 0, 0)),
            pl.BlockSpec((5, _LB), lambda i: (0, i)),
            pl.BlockSpec((5, _LB), lambda i: (0, i)),
        ],
        out_specs=pl.BlockSpec((_B, _MAX_DIS, _LB), lambda i: (0, 0, i)),
        compiler_params=pltpu.CompilerParams(
            dimension_semantics=("arbitrary",),
        ),
        name="cdist_count",
    )(ah, al, lh, ll)
    return res.transpose(0, 2, 1)


# speedup vs baseline: 2.4472x; 1.0908x over previous
import jax
import jax.numpy as jnp
from jax.experimental import pallas as pl
from jax.experimental.pallas import tpu as pltpu

_MAX_DIS = 15
_B = 4
_N = 8192
_L = 2048
_LB = 128
_PS = 128
_FLUSH = 16


def _cdist_count_kernel(ah_ref, al_ref, lh_ref, ll_ref, out_ref):
    lh = lh_ref[...]
    ll = ll_ref[...]
    one = jnp.bfloat16(1.0)
    zero = jnp.bfloat16(0.0)
    dims = (((0,), (0,)), ((), ()))
    n_seg = _N // _PS // _FLUSH

    n_chunks = _N // _PS
    for b in range(_B):
        accs = [jnp.zeros((1, _LB), jnp.float32) for _ in range(_MAX_DIS)]
        acc16 = [jnp.zeros((16, _LB), jnp.bfloat16) for _ in range(_MAX_DIS)]
        for c in range(n_chunks):
            pah = ah_ref[b, :, c * _PS:(c + 1) * _PS]
            pal = al_ref[b, :, c * _PS:(c + 1) * _PS]
            d2 = (
                jax.lax.dot_general(pah, lh, dims,
                                    preferred_element_type=jnp.float32)
                + jax.lax.dot_general(pah, ll, dims,
                                      preferred_element_type=jnp.float32)
                + jax.lax.dot_general(pal, lh, dims,
                                      preferred_element_type=jnp.float32)
            )
            e16 = jnp.ceil(d2).astype(jnp.bfloat16)
            for t in range(_MAX_DIS):
                thr2 = jnp.bfloat16((t + 1) * (t + 1))
                m = jnp.where(e16 <= thr2, one, zero)
                m = m[0:64] + m[64:128]
                m = m[0:32] + m[32:64]
                m = m[0:16] + m[16:32]
                acc16[t] = acc16[t] + m
            if c % _FLUSH == _FLUSH - 1:
                for t in range(_MAX_DIS):
                    s = jnp.sum(acc16[t].astype(jnp.float32), axis=0,
                                keepdims=True)
                    accs[t] = accs[t] + s
                    acc16[t] = jnp.zeros((16, _LB), jnp.bfloat16)
        for t in range(_MAX_DIS):
            out_ref[b, t:t + 1, :] = accs[t]


def _trunc_bf16_exact(x):
    u = jax.lax.bitcast_convert_type(x, jnp.uint32)
    return jax.lax.bitcast_convert_type(u & jnp.uint32(0xFFFF0000),
                                        jnp.float32)


def _split_bf16_exact(x):
    hi = _trunc_bf16_exact(x)
    lo = _trunc_bf16_exact(x - hi)
    return hi, lo


def kernel(pcd, locs):
    pn2 = jnp.sum(pcd * pcd, axis=-1, keepdims=True)
    aug = jnp.concatenate(
        [pcd, pn2, jnp.ones_like(pn2)], axis=-1
    ).transpose(0, 2, 1)
    cl2 = jnp.sum(locs * locs, axis=-1)[None, :]
    laug = jnp.concatenate(
        [-2.0 * locs.T, jnp.ones_like(cl2), cl2], axis=0
    )
    ah, al = _split_bf16_exact(aug)
    lh, ll = _split_bf16_exact(laug)
    res = pl.pallas_call(
        _cdist_count_kernel,
        out_shape=jax.ShapeDtypeStruct((_B, _MAX_DIS, _L), jnp.float32),
        grid=(_L // _LB,),
        in_specs=[
            pl.BlockSpec((_B, 5, _N), lambda i: (0, 0, 0)),
            pl.BlockSpec((_B, 5, _N), lambda i: (0, 0, 0)),
            pl.BlockSpec((5, _LB), lambda i: (0, i)),
            pl.BlockSpec((5, _LB), lambda i: (0, i)),
        ],
        out_specs=pl.BlockSpec((_B, _MAX_DIS, _LB), lambda i: (0, 0, i)),
        compiler_params=pltpu.CompilerParams(
            dimension_semantics=("arbitrary",),
        ),
        name="cdist_count",
    )(ah, al, lh, ll)
    return res.transpose(0, 2, 1)


# speedup vs baseline: 2.4482x; 1.0004x over previous
import jax
import jax.numpy as jnp
from jax.experimental import pallas as pl
from jax.experimental.pallas import tpu as pltpu

_MAX_DIS = 15
_B = 4
_N = 8192
_L = 2048
_LB = 128
_PS = 128
_FLUSH = 16


def _cdist_count_kernel(ah_ref, al_ref, lh_ref, ll_ref, out_ref):
    lh = lh_ref[...]
    ll = ll_ref[...]
    one = jnp.bfloat16(1.0)
    zero = jnp.bfloat16(0.0)
    dims = (((0,), (0,)), ((), ()))
    n_chunks = _N // _PS
    for b in range(_B):
        accs = [jnp.zeros((1, _LB), jnp.float32) for _ in range(_MAX_DIS)]
        acc16 = [jnp.zeros((16, _LB), jnp.bfloat16) for _ in range(_MAX_DIS)]
        for c in range(n_chunks):
            pah = ah_ref[b, :, c * _PS:(c + 1) * _PS]
            pal = al_ref[b, :, c * _PS:(c + 1) * _PS]
            d2 = (
                jax.lax.dot_general(pah, lh, dims,
                                    preferred_element_type=jnp.float32)
                + jax.lax.dot_general(pah, ll, dims,
                                      preferred_element_type=jnp.float32)
                + jax.lax.dot_general(pal, lh, dims,
                                      preferred_element_type=jnp.float32)
            )
            e16 = jnp.ceil(d2).astype(jnp.bfloat16)
            for t in range(_MAX_DIS):
                thr2 = jnp.bfloat16((t + 1) * (t + 1))
                m = jnp.where(e16 <= thr2, one, zero)
                m = m[0:64] + m[64:128]
                m = m[0:32] + m[32:64]
                m = m[0:16] + m[16:32]
                acc16[t] = acc16[t] + m
            if c % _FLUSH == _FLUSH - 1:
                for t in range(_MAX_DIS):
                    s = jnp.sum(acc16[t].astype(jnp.float32), axis=0,
                                keepdims=True)
                    accs[t] = accs[t] + s
                    acc16[t] = jnp.zeros((16, _LB), jnp.bfloat16)
        for t in range(_MAX_DIS):
            out_ref[b, t:t + 1, :] = accs[t]


def _trunc_bf16_exact(x):
    u = jax.lax.bitcast_convert_type(x, jnp.uint32)
    return jax.lax.bitcast_convert_type(u & jnp.uint32(0xFFFF0000),
                                        jnp.float32)


def _split_bf16_exact(x):
    hi = _trunc_bf16_exact(x)
    lo = _trunc_bf16_exact(x - hi)
    return hi, lo


def kernel(pcd, locs):
    pn2 = jnp.sum(pcd * pcd, axis=-1, keepdims=True)
    aug = jnp.concatenate(
        [pcd, pn2, jnp.ones_like(pn2)], axis=-1
    ).transpose(0, 2, 1)
    cl2 = jnp.sum(locs * locs, axis=-1)[None, :]
    laug = jnp.concatenate(
        [-2.0 * locs.T, jnp.ones_like(cl2), cl2], axis=0
    )
    ah, al = _split_bf16_exact(aug)
    lh, ll = _split_bf16_exact(laug)
    res = pl.pallas_call(
        _cdist_count_kernel,
        out_shape=jax.ShapeDtypeStruct((_B, _MAX_DIS, _L), jnp.float32),
        grid=(_L // _LB,),
        in_specs=[
            pl.BlockSpec((_B, 5, _N), lambda i: (0, 0, 0)),
            pl.BlockSpec((_B, 5, _N), lambda i: (0, 0, 0)),
            pl.BlockSpec((5, _LB), lambda i: (0, i)),
            pl.BlockSpec((5, _LB), lambda i: (0, i)),
        ],
        out_specs=pl.BlockSpec((_B, _MAX_DIS, _LB), lambda i: (0, 0, i)),
        compiler_params=pltpu.CompilerParams(
            dimension_semantics=("arbitrary",),
        ),
        name="cdist_count",
    )(ah, al, lh, ll)
    return res.transpose(0, 2, 1)
